# baseline (speedup 1.0000x reference)
import functools

import jax
import jax.numpy as jnp
from jax import lax
from jax.experimental import pallas as pl
from jax.experimental.pallas import tpu as pltpu
from jax.experimental.pallas import tpu_sc as plsc

NN = 10000
EE = 320000
DD = 128
GG = 64

NC = 2
NS = 16
NW = NC * NS

CHUNK = 128
EPT = -(-EE // NW)
CPT = -(-EPT // CHUNK)
EPTP = CPT * CHUNK
EPAD = NW * EPTP

C0 = 80
C1 = 80
TOT = NS * (C0 + C1)
EPAD2 = TOT * CHUNK

NPAD = 10112
RPT = NPAD // NS

RB = 400
GRID = NN // RB

_sc_mesh = plsc.VectorSubcoreMesh(core_axis_name="c", subcore_axis_name="s")


@functools.partial(
    pl.kernel,
    out_type=jax.ShapeDtypeStruct((NC, NPAD, DD), jnp.float32),
    mesh=_sc_mesh,
    scratch_types=[
        pltpu.VMEM((CPT, CHUNK), jnp.int32),
        pltpu.VMEM((CHUNK, DD), jnp.float32),
        pltpu.VMEM((CHUNK, DD), jnp.float32),
        pltpu.VMEM_SHARED((NPAD, DD), jnp.float32),
    ],
)
def _sc_deg(dst_hbm, ones_hbm, zero_hbm, out_hbm, dst_v, ones_v, zero_v, acc):
    c = lax.axis_index("c")
    s = lax.axis_index("s")
    wid = c * NS + s
    pltpu.sync_copy(dst_hbm.at[wid], dst_v)
    pltpu.sync_copy(ones_hbm, ones_v)
    pltpu.sync_copy(zero_hbm, zero_v)
    base = s * RPT
    for k in range(RPT // CHUNK):
        pltpu.sync_copy(zero_v, acc.at[pl.ds(base + k * CHUNK, CHUNK)])
    rem = RPT % CHUNK
    if rem:
        pltpu.sync_copy(zero_v.at[pl.ds(0, rem)],
                        acc.at[pl.ds(base + (RPT // CHUNK) * CHUNK, rem)])
    plsc.subcore_barrier()

    def body(j, carry):
        pltpu.sync_copy(ones_v, acc.at[dst_v.at[j]], add=True)
        return carry

    lax.fori_loop(0, CPT, body, 0)
    plsc.subcore_barrier()
    pltpu.sync_copy(acc.at[pl.ds(base, RPT)], out_hbm.at[c, pl.ds(base, RPT)])


@functools.partial(
    pl.kernel,
    out_type=jax.ShapeDtypeStruct((NC, NPAD, DD), jnp.float32),
    mesh=_sc_mesh,
    scratch_types=[
        pltpu.VMEM((4, 2, CHUNK), jnp.int32),
        pltpu.VMEM((CHUNK, DD), jnp.float32),
        pltpu.VMEM((CHUNK, DD), jnp.float32),
        pltpu.VMEM_SHARED((NPAD, DD), jnp.float32),
        pltpu.SemaphoreType.DMA,
        pltpu.SemaphoreType.DMA,
        pltpu.SemaphoreType.DMA,
        pltpu.SemaphoreType.DMA,
        pltpu.SemaphoreType.DMA,
        pltpu.SemaphoreType.DMA,
    ],
)
def _sc_agg(y_hbm, ei_hbm, zero_hbm, out_hbm, ring_v, rows_a, rows_b, acc,
            is0, is1, is2, is3, ga, gb):
    c = lax.axis_index("c")
    s = lax.axis_index("s")
    start = jnp.where(c == 0, s * C0, NS * C0 + s * C1)
    cnt = jnp.where(c == 0, C0, C1)
    isems = [is0, is1, is2, is3]
    rows = [rows_a, rows_b]
    gsems = [ga, gb]

    def idx_start(chunk, slot, sem):
        pltpu.async_copy(ei_hbm.at[start + chunk], ring_v.at[slot], sem)

    def idx_wait(chunk, slot, sem):
        pltpu.make_async_copy(
            ei_hbm.at[start + chunk], ring_v.at[slot], sem).wait()

    def g_start(slot, buf, sem):
        pltpu.async_copy(y_hbm.at[ring_v.at[slot, 0]], buf, sem)

    def g_wait(slot, buf, sem):
        pltpu.make_async_copy(y_hbm.at[ring_v.at[slot, 0]], buf, sem).wait()

    def scat(slot, buf):
        pltpu.sync_copy(buf, acc.at[ring_v.at[slot, 1]], add=True)

    pltpu.sync_copy(zero_hbm, rows_a)
    base = s * RPT
    for k in range(RPT // CHUNK):
        pltpu.sync_copy(rows_a, acc.at[pl.ds(base + k * CHUNK, CHUNK)])
    rem = RPT % CHUNK
    if rem:
        pltpu.sync_copy(rows_a.at[pl.ds(0, rem)],
                        acc.at[pl.ds(base + (RPT // CHUNK) * CHUNK, rem)])
    plsc.subcore_barrier()

    for k in range(4):
        idx_start(k, k, isems[k])
    idx_wait(0, 0, isems[0])
    g_start(0, rows_a, ga)

    def body(j2, carry):
        j = j2 * 4
        for u in range(4):
            cu = j + u
            nslot = (u + 1) % 4
            nbuf = (u + 1) % 2
            if u < 3:
                idx_wait(cu + 1, nslot, isems[nslot])
                g_start(nslot, rows[nbuf], gsems[nbuf])
            else:
                @pl.when(cu + 1 < cnt)
                def _():
                    idx_wait(cu + 1, nslot, isems[nslot])
                    g_start(nslot, rows[nbuf], gsems[nbuf])
            g_wait(u, rows[u % 2], gsems[u % 2])
            scat(u, rows[u % 2])
            @pl.when(cu + 4 < cnt)
            def _():
                idx_start(cu + 4, u, isems[u])
        return carry

    lax.fori_loop(0, cnt // 4, body, 0)
    plsc.subcore_barrier()
    pltpu.sync_copy(acc.at[pl.ds(base, RPT)], out_hbm.at[c, pl.ds(base, RPT)])


def _sig(v):
    return 1.0 / (1.0 + jnp.exp(-v))


def _tc_dinv_body(deg_ref, out_ref):
    d = deg_ref[0, :, 0:1] + deg_ref[1, :, 0:1] + 1.0
    dv = lax.rsqrt(d)
    out_ref[...] = jnp.broadcast_to(dv, (RB, DD))


_tc_dinv = pl.pallas_call(
    _tc_dinv_body,
    grid=(GRID,),
    in_specs=[pl.BlockSpec((NC, RB, DD), lambda i: (0, i, 0))],
    out_specs=pl.BlockSpec((RB, DD), lambda i: (i, 0)),
    out_shape=jax.ShapeDtypeStruct((NN, DD), jnp.float32),
)


def _tc_prep1_body(x_ref, dinv_ref, w_ref, b_ref, y_ref, s_ref):
    xw = jnp.dot(x_ref[...], w_ref[...], preferred_element_type=jnp.float32)
    dv = dinv_ref[...]
    y_ref[...] = dv * xw
    s_ref[...] = dv * dv * xw + b_ref[...]


_tc_prep1 = pl.pallas_call(
    _tc_prep1_body,
    grid=(GRID,),
    in_specs=[
        pl.BlockSpec((RB, DD), lambda i: (i, 0)),
        pl.BlockSpec((RB, DD), lambda i: (i, 0)),
        pl.BlockSpec((DD, DD), lambda i: (0, 0)),
        pl.BlockSpec((1, DD), lambda i: (0, 0)),
    ],
    out_specs=[
        pl.BlockSpec((RB, DD), lambda i: (i, 0)),
        pl.BlockSpec((RB, DD), lambda i: (i, 0)),
    ],
    out_shape=[
        jax.ShapeDtypeStruct((NN, DD), jnp.float32),
        jax.ShapeDtypeStruct((NN, DD), jnp.float32),
    ],
)


def _tc_mid_body(agg_ref, s_ref, dinv_ref, w_ref, b_ref, y_ref, s2_ref):
    dv = dinv_ref[...]
    h = _sig(dv * (agg_ref[0] + agg_ref[1]) + s_ref[...])
    xw = jnp.dot(h, w_ref[...], preferred_element_type=jnp.float32)
    y_ref[...] = dv * xw
    s2_ref[...] = dv * dv * xw + b_ref[...]


_tc_mid = pl.pallas_call(
    _tc_mid_body,
    grid=(GRID,),
    in_specs=[
        pl.BlockSpec((NC, RB, DD), lambda i: (0, i, 0)),
        pl.BlockSpec((RB, DD), lambda i: (i, 0)),
        pl.BlockSpec((RB, DD), lambda i: (i, 0)),
        pl.BlockSpec((DD, DD), lambda i: (0, 0)),
        pl.BlockSpec((1, DD), lambda i: (0, 0)),
    ],
    out_specs=[
        pl.BlockSpec((RB, DD), lambda i: (i, 0)),
        pl.BlockSpec((RB, DD), lambda i: (i, 0)),
    ],
    out_shape=[
        jax.ShapeDtypeStruct((NN, DD), jnp.float32),
        jax.ShapeDtypeStruct((NN, DD), jnp.float32),
    ],
)


def _tc_final_body(agg_ref, s_ref, dinv_ref, batch_ref, wc1_ref, bc1_ref,
                   wc2_ref, bc2_ref, out_ref, sums_ref, cnts_ref):
    i = pl.program_id(0)

    @pl.when(i == 0)
    def _():
        sums_ref[...] = jnp.zeros((GG, DD), jnp.float32)
        cnts_ref[...] = jnp.zeros((GG, DD), jnp.float32)
        out_ref[...] = jnp.zeros((GG, 1), jnp.float32)

    dv = dinv_ref[...]
    h = _sig(dv * (agg_ref[0] + agg_ref[1]) + s_ref[...])
    ids = batch_ref[0, 0, :]
    oh = (ids[None, :] == lax.broadcasted_iota(jnp.int32, (GG, RB), 0))
    oh = oh.astype(jnp.float32)
    sums_ref[...] += jnp.dot(oh, h, preferred_element_type=jnp.float32)
    cnts_ref[...] += jnp.broadcast_to(
        jnp.sum(oh, axis=1, keepdims=True), (GG, DD))

    @pl.when(i == GRID - 1)
    def _():
        pooled = sums_ref[...] / jnp.maximum(cnts_ref[...], 1.0)
        hc = _sig(jnp.dot(pooled, wc1_ref[...],
                          preferred_element_type=jnp.float32) + bc1_ref[...])
        out_ref[...] = jnp.dot(hc, wc2_ref[...],
                               preferred_element_type=jnp.float32) + bc2_ref[...]


_tc_final = pl.pallas_call(
    _tc_final_body,
    grid=(GRID,),
    in_specs=[
        pl.BlockSpec((NC, RB, DD), lambda i: (0, i, 0)),
        pl.BlockSpec((RB, DD), lambda i: (i, 0)),
        pl.BlockSpec((RB, DD), lambda i: (i, 0)),
        pl.BlockSpec((1, 1, RB), lambda i: (i, 0, 0)),
        pl.BlockSpec((DD, GG), lambda i: (0, 0)),
        pl.BlockSpec((1, GG), lambda i: (0, 0)),
        pl.BlockSpec((GG, 1), lambda i: (0, 0)),
        pl.BlockSpec((1, 1), lambda i: (0, 0)),
    ],
    out_specs=pl.BlockSpec((GG, 1), lambda i: (0, 0)),
    out_shape=jax.ShapeDtypeStruct((GG, 1), jnp.float32),
    scratch_shapes=[
        pltpu.VMEM((GG, DD), jnp.float32),
        pltpu.VMEM((GG, DD), jnp.float32),
    ],
)


def kernel(x, edge_index, batch, num_graphs, W1, b1, W2, b2, Wc1, bc1, Wc2, bc2):
    src = edge_index[0]
    dst = edge_index[1]
    npad = EPAD - EE
    dst3 = jnp.concatenate(
        [dst, jnp.full((npad,), NN, jnp.int32)]).reshape(NW, CPT, CHUNK)
    npad2 = EPAD2 - EE
    srcT = jnp.concatenate(
        [src, jnp.zeros((npad2,), jnp.int32)]).reshape(TOT, CHUNK)
    dstT = jnp.concatenate(
        [dst, jnp.full((npad2,), NN, jnp.int32)]).reshape(TOT, CHUNK)
    ei4 = jnp.stack([srcT, dstT], axis=1)
    batch3 = batch.reshape(GRID, 1, RB)
    ones_row = jnp.ones((CHUNK, DD), jnp.float32)
    zero_row = jnp.zeros((CHUNK, DD), jnp.float32)

    deg2 = _sc_deg(dst3, ones_row, zero_row)
    dinvb = _tc_dinv(deg2)
    y1, s1 = _tc_prep1(x, dinvb, W1, b1.reshape(1, DD))
    agg1 = _sc_agg(y1, ei4, zero_row)
    y2, s2 = _tc_mid(agg1, s1, dinvb, W2, b2.reshape(1, DD))
    agg2 = _sc_agg(y2, ei4, zero_row)
    out = _tc_final(agg2, s2, dinvb, batch3, Wc1, bc1.reshape(1, GG),
                    Wc2, bc2.reshape(1, 1))
    return out


# speedup vs baseline: 1.4154x; 1.4154x over previous
import functools

import jax
import jax.numpy as jnp
from jax import lax
from jax.experimental import pallas as pl
from jax.experimental.pallas import tpu as pltpu
from jax.experimental.pallas import tpu_sc as plsc

NN = 10000
EE = 320000
DD = 128
GG = 64

NC = 2
NS = 16
NW = NC * NS

CHUNK = 128
EPT = -(-EE // NW)
CPT = -(-EPT // CHUNK)
EPTP = CPT * CHUNK
EPAD = NW * EPTP

NPAD = 10112
RPT = NPAD // NS

RB = 400
GRID = NN // RB

_sc_mesh = plsc.VectorSubcoreMesh(core_axis_name="c", subcore_axis_name="s")


@functools.partial(
    pl.kernel,
    out_type=jax.ShapeDtypeStruct((NC, NPAD, DD), jnp.float32),
    mesh=_sc_mesh,
    scratch_types=[
        pltpu.VMEM((CPT, CHUNK), jnp.int32),
        pltpu.VMEM((CHUNK, DD), jnp.float32),
        pltpu.VMEM((CHUNK, DD), jnp.float32),
        pltpu.VMEM_SHARED((NPAD, DD), jnp.float32),
    ],
)
def _sc_deg(dst_hbm, ones_hbm, zero_hbm, out_hbm, dst_v, ones_v, zero_v, acc):
    c = lax.axis_index("c")
    s = lax.axis_index("s")
    wid = c * NS + s
    pltpu.sync_copy(dst_hbm.at[wid], dst_v)
    pltpu.sync_copy(ones_hbm, ones_v)
    pltpu.sync_copy(zero_hbm, zero_v)
    base = s * RPT
    for k in range(RPT // CHUNK):
        pltpu.sync_copy(zero_v, acc.at[pl.ds(base + k * CHUNK, CHUNK)])
    rem = RPT % CHUNK
    if rem:
        pltpu.sync_copy(zero_v.at[pl.ds(0, rem)],
                        acc.at[pl.ds(base + (RPT // CHUNK) * CHUNK, rem)])
    plsc.subcore_barrier()

    def body(j, carry):
        pltpu.sync_copy(ones_v, acc.at[dst_v.at[j]], add=True)
        return carry

    lax.fori_loop(0, CPT, body, 0)
    plsc.subcore_barrier()
    pltpu.sync_copy(acc.at[pl.ds(base, RPT)], out_hbm.at[c, pl.ds(base, RPT)])


@functools.partial(
    pl.kernel,
    out_type=jax.ShapeDtypeStruct((NC, NPAD, DD), jnp.float32),
    mesh=_sc_mesh,
    scratch_types=[
        pltpu.VMEM((4, 2, CHUNK), jnp.int32),
        pltpu.VMEM((CHUNK, DD), jnp.float32),
        pltpu.VMEM((CHUNK, DD), jnp.float32),
        pltpu.VMEM_SHARED((NPAD, DD), jnp.float32),
        pltpu.SemaphoreType.DMA,
        pltpu.SemaphoreType.DMA,
        pltpu.SemaphoreType.DMA,
        pltpu.SemaphoreType.DMA,
        pltpu.SemaphoreType.DMA,
        pltpu.SemaphoreType.DMA,
    ],
)
def _sc_agg(y_hbm, ei_hbm, zero_hbm, out_hbm, ring_v, rows_a, rows_b, acc,
            is0, is1, is2, is3, ga, gb):
    c = lax.axis_index("c")
    s = lax.axis_index("s")
    wid = c * NS + s
    isems = [is0, is1, is2, is3]
    rows = [rows_a, rows_b]
    gsems = [ga, gb]

    def idx_start(chunk, slot, sem):
        pltpu.async_copy(ei_hbm.at[wid, chunk], ring_v.at[slot], sem)

    def idx_wait(chunk, slot, sem):
        pltpu.make_async_copy(
            ei_hbm.at[wid, chunk], ring_v.at[slot], sem).wait()

    def g_start(slot, buf, sem):
        pltpu.async_copy(y_hbm.at[ring_v.at[slot, 0]], buf, sem)

    def g_wait(slot, buf, sem):
        pltpu.make_async_copy(y_hbm.at[ring_v.at[slot, 0]], buf, sem).wait()

    def scat(slot, buf):
        pltpu.sync_copy(buf, acc.at[ring_v.at[slot, 1]], add=True)

    pltpu.sync_copy(zero_hbm, rows_a)
    base = s * RPT
    for k in range(RPT // CHUNK):
        pltpu.sync_copy(rows_a, acc.at[pl.ds(base + k * CHUNK, CHUNK)])
    rem = RPT % CHUNK
    if rem:
        pltpu.sync_copy(rows_a.at[pl.ds(0, rem)],
                        acc.at[pl.ds(base + (RPT // CHUNK) * CHUNK, rem)])
    plsc.subcore_barrier()

    L = CPT // 4
    R = CPT - 4 * L
    for k in range(4):
        idx_start(k, k, isems[k])
    idx_wait(0, 0, isems[0])
    g_start(0, rows_a, ga)

    def body(j2, carry):
        j = j2 * 4
        for u in range(4):
            cu = j + u
            nslot = (u + 1) % 4
            nbuf = (u + 1) % 2
            if u < 3:
                idx_wait(cu + 1, nslot, isems[nslot])
                g_start(nslot, rows[nbuf], gsems[nbuf])
            else:
                @pl.when(cu + 1 < CPT)
                def _():
                    idx_wait(cu + 1, nslot, isems[nslot])
                    g_start(nslot, rows[nbuf], gsems[nbuf])
            g_wait(u, rows[u % 2], gsems[u % 2])
            scat(u, rows[u % 2])
            @pl.when(cu + 4 < CPT)
            def _():
                idx_start(cu + 4, u, isems[u])
        return carry

    lax.fori_loop(0, L, body, 0)
    for k in range(R):
        cu = 4 * L + k
        if k + 1 < R:
            idx_wait(cu + 1, k + 1, isems[k + 1])
            g_start(k + 1, rows[(k + 1) % 2], gsems[(k + 1) % 2])
        g_wait(k, rows[k % 2], gsems[k % 2])
        scat(k, rows[k % 2])
    plsc.subcore_barrier()
    pltpu.sync_copy(acc.at[pl.ds(base, RPT)], out_hbm.at[c, pl.ds(base, RPT)])


def _sig(v):
    return 1.0 / (1.0 + jnp.exp(-v))


def _tc_prep1_body(x_ref, deg_ref, w_ref, b_ref, y_ref, s_ref, dinv_ref):
    d = deg_ref[0, :, 0:1] + deg_ref[1, :, 0:1] + 1.0
    dv = lax.rsqrt(d)
    xw = jnp.dot(x_ref[...], w_ref[...], preferred_element_type=jnp.float32)
    y_ref[...] = dv * xw
    s_ref[...] = dv * dv * xw + b_ref[...]
    dinv_ref[...] = jnp.broadcast_to(dv, (RB, DD))


_tc_prep1 = pl.pallas_call(
    _tc_prep1_body,
    grid=(GRID,),
    in_specs=[
        pl.BlockSpec((RB, DD), lambda i: (i, 0)),
        pl.BlockSpec((NC, RB, DD), lambda i: (0, i, 0)),
        pl.BlockSpec((DD, DD), lambda i: (0, 0)),
        pl.BlockSpec((1, DD), lambda i: (0, 0)),
    ],
    out_specs=[
        pl.BlockSpec((RB, DD), lambda i: (i, 0)),
        pl.BlockSpec((RB, DD), lambda i: (i, 0)),
        pl.BlockSpec((RB, DD), lambda i: (i, 0)),
    ],
    out_shape=[
        jax.ShapeDtypeStruct((NN, DD), jnp.float32),
        jax.ShapeDtypeStruct((NN, DD), jnp.float32),
        jax.ShapeDtypeStruct((NN, DD), jnp.float32),
    ],
)


def _tc_mid_body(agg_ref, s_ref, dinv_ref, w_ref, b_ref, y_ref, s2_ref):
    dv = dinv_ref[...]
    h = _sig(dv * (agg_ref[0] + agg_ref[1]) + s_ref[...])
    xw = jnp.dot(h, w_ref[...], preferred_element_type=jnp.float32)
    y_ref[...] = dv * xw
    s2_ref[...] = dv * dv * xw + b_ref[...]


_tc_mid = pl.pallas_call(
    _tc_mid_body,
    grid=(GRID,),
    in_specs=[
        pl.BlockSpec((NC, RB, DD), lambda i: (0, i, 0)),
        pl.BlockSpec((RB, DD), lambda i: (i, 0)),
        pl.BlockSpec((RB, DD), lambda i: (i, 0)),
        pl.BlockSpec((DD, DD), lambda i: (0, 0)),
        pl.BlockSpec((1, DD), lambda i: (0, 0)),
    ],
    out_specs=[
        pl.BlockSpec((RB, DD), lambda i: (i, 0)),
        pl.BlockSpec((RB, DD), lambda i: (i, 0)),
    ],
    out_shape=[
        jax.ShapeDtypeStruct((NN, DD), jnp.float32),
        jax.ShapeDtypeStruct((NN, DD), jnp.float32),
    ],
)


def _tc_final_body(agg_ref, s_ref, dinv_ref, batch_ref, wc1_ref, bc1_ref,
                   wc2_ref, bc2_ref, out_ref, sums_ref, cnts_ref):
    i = pl.program_id(0)

    @pl.when(i == 0)
    def _():
        sums_ref[...] = jnp.zeros((GG, DD), jnp.float32)
        cnts_ref[...] = jnp.zeros((GG, DD), jnp.float32)
        out_ref[...] = jnp.zeros((GG, 1), jnp.float32)

    dv = dinv_ref[...]
    h = _sig(dv * (agg_ref[0] + agg_ref[1]) + s_ref[...])
    ids = batch_ref[0, 0, :]
    oh = (ids[None, :] == lax.broadcasted_iota(jnp.int32, (GG, RB), 0))
    oh = oh.astype(jnp.float32)
    sums_ref[...] += jnp.dot(oh, h, preferred_element_type=jnp.float32)
    cnts_ref[...] += jnp.broadcast_to(
        jnp.sum(oh, axis=1, keepdims=True), (GG, DD))

    @pl.when(i == GRID - 1)
    def _():
        pooled = sums_ref[...] / jnp.maximum(cnts_ref[...], 1.0)
        hc = _sig(jnp.dot(pooled, wc1_ref[...],
                          preferred_element_type=jnp.float32) + bc1_ref[...])
        out_ref[...] = jnp.dot(hc, wc2_ref[...],
                               preferred_element_type=jnp.float32) + bc2_ref[...]


_tc_final = pl.pallas_call(
    _tc_final_body,
    grid=(GRID,),
    in_specs=[
        pl.BlockSpec((NC, RB, DD), lambda i: (0, i, 0)),
        pl.BlockSpec((RB, DD), lambda i: (i, 0)),
        pl.BlockSpec((RB, DD), lambda i: (i, 0)),
        pl.BlockSpec((1, 1, RB), lambda i: (i, 0, 0)),
        pl.BlockSpec((DD, GG), lambda i: (0, 0)),
        pl.BlockSpec((1, GG), lambda i: (0, 0)),
        pl.BlockSpec((GG, 1), lambda i: (0, 0)),
        pl.BlockSpec((1, 1), lambda i: (0, 0)),
    ],
    out_specs=pl.BlockSpec((GG, 1), lambda i: (0, 0)),
    out_shape=jax.ShapeDtypeStruct((GG, 1), jnp.float32),
    scratch_shapes=[
        pltpu.VMEM((GG, DD), jnp.float32),
        pltpu.VMEM((GG, DD), jnp.float32),
    ],
)


def kernel(x, edge_index, batch, num_graphs, W1, b1, W2, b2, Wc1, bc1, Wc2, bc2):
    src = edge_index[0]
    dst = edge_index[1]
    npad = EPAD - EE
    src3 = jnp.concatenate(
        [src, jnp.zeros((npad,), jnp.int32)]).reshape(NW, CPT, CHUNK)
    dst3 = jnp.concatenate(
        [dst, jnp.full((npad,), NN, jnp.int32)]).reshape(NW, CPT, CHUNK)
    ei4 = jnp.stack([src3, dst3], axis=2)
    batch3 = batch.reshape(GRID, 1, RB)
    ones_row = jnp.ones((CHUNK, DD), jnp.float32)
    zero_row = jnp.zeros((CHUNK, DD), jnp.float32)

    deg2 = _sc_deg(dst3, ones_row, zero_row)
    y1, s1, dinvb = _tc_prep1(x, deg2, W1, b1.reshape(1, DD))
    agg1 = _sc_agg(y1, ei4, zero_row)
    y2, s2 = _tc_mid(agg1, s1, dinvb, W2, b2.reshape(1, DD))
    agg2 = _sc_agg(y2, ei4, zero_row)
    out = _tc_final(agg2, s2, dinvb, batch3, Wc1, bc1.reshape(1, GG),
                    Wc2, bc2.reshape(1, 1))
    return out


# speedup vs baseline: 1.4967x; 1.0575x over previous
import functools

import jax
import jax.numpy as jnp
from jax import lax
from jax.experimental import pallas as pl
from jax.experimental.pallas import tpu as pltpu
from jax.experimental.pallas import tpu_sc as plsc

NN = 10000
EE = 320000
DD = 128
GG = 64

NC = 2
NS = 16
NW = NC * NS

CHUNK = 128
EPT = -(-EE // NW)
CPT = -(-EPT // CHUNK)
EPTP = CPT * CHUNK
EPAD = NW * EPTP

NPAD = 10112
RPT = NPAD // NS

RB = 400
GRID = NN // RB

_sc_mesh = plsc.VectorSubcoreMesh(core_axis_name="c", subcore_axis_name="s")


@functools.partial(
    pl.kernel,
    out_type=jax.ShapeDtypeStruct((NC, NPAD, DD), jnp.float32),
    mesh=_sc_mesh,
    scratch_types=[
        pltpu.VMEM((CPT, CHUNK), jnp.int32),
        pltpu.VMEM((CHUNK, DD), jnp.float32),
        pltpu.VMEM((CHUNK, DD), jnp.float32),
        pltpu.VMEM_SHARED((NPAD, DD), jnp.float32),
    ],
)
def _sc_deg(dst_hbm, ones_hbm, zero_hbm, out_hbm, dst_v, ones_v, zero_v, acc):
    c = lax.axis_index("c")
    s = lax.axis_index("s")
    wid = c * NS + s
    pltpu.sync_copy(dst_hbm.at[wid], dst_v)
    pltpu.sync_copy(ones_hbm, ones_v)
    pltpu.sync_copy(zero_hbm, zero_v)
    base = s * RPT
    for k in range(RPT // CHUNK):
        pltpu.sync_copy(zero_v, acc.at[pl.ds(base + k * CHUNK, CHUNK)])
    rem = RPT % CHUNK
    if rem:
        pltpu.sync_copy(zero_v.at[pl.ds(0, rem)],
                        acc.at[pl.ds(base + (RPT // CHUNK) * CHUNK, rem)])
    plsc.subcore_barrier()

    def body(j, carry):
        pltpu.sync_copy(ones_v, acc.at[dst_v.at[j]], add=True)
        return carry

    lax.fori_loop(0, CPT, body, 0)
    plsc.subcore_barrier()
    pltpu.sync_copy(acc.at[pl.ds(base, RPT)], out_hbm.at[c, pl.ds(base, RPT)])


@functools.partial(
    pl.kernel,
    out_type=jax.ShapeDtypeStruct((NC, NPAD, DD), jnp.float32),
    mesh=_sc_mesh,
    scratch_types=[
        pltpu.VMEM((4, 2, CHUNK), jnp.int32),
        pltpu.VMEM((CHUNK, DD), jnp.float32),
        pltpu.VMEM((CHUNK, DD), jnp.float32),
        pltpu.VMEM_SHARED((NPAD, DD), jnp.float32),
        pltpu.SemaphoreType.DMA,
        pltpu.SemaphoreType.DMA,
        pltpu.SemaphoreType.DMA,
        pltpu.SemaphoreType.DMA,
        pltpu.SemaphoreType.DMA,
        pltpu.SemaphoreType.DMA,
    ],
)
def _sc_agg(y_hbm, ei_hbm, zero_hbm, out_hbm, ring_v, rows_a, rows_b, acc,
            is0, is1, is2, is3, ga, gb):
    c = lax.axis_index("c")
    s = lax.axis_index("s")
    wid = c * NS + s
    isems = [is0, is1, is2, is3]
    rows = [rows_a, rows_b]
    gsems = [ga, gb]

    def idx_start(chunk, slot, sem):
        pltpu.async_copy(ei_hbm.at[wid, chunk], ring_v.at[slot], sem)

    def idx_wait(chunk, slot, sem):
        pltpu.make_async_copy(
            ei_hbm.at[wid, chunk], ring_v.at[slot], sem).wait()

    def g_start(slot, buf, sem):
        pltpu.async_copy(y_hbm.at[ring_v.at[slot, 0]], buf, sem)

    def g_wait(slot, buf, sem):
        pltpu.make_async_copy(y_hbm.at[ring_v.at[slot, 0]], buf, sem).wait()

    def scat(slot, buf):
        pltpu.sync_copy(buf, acc.at[ring_v.at[slot, 1]], add=True)

    pltpu.sync_copy(zero_hbm, rows_a)
    base = s * RPT
    for k in range(RPT // CHUNK):
        pltpu.sync_copy(rows_a, acc.at[pl.ds(base + k * CHUNK, CHUNK)])
    rem = RPT % CHUNK
    if rem:
        pltpu.sync_copy(rows_a.at[pl.ds(0, rem)],
                        acc.at[pl.ds(base + (RPT // CHUNK) * CHUNK, rem)])
    plsc.subcore_barrier()

    L = CPT // 4
    R = CPT - 4 * L
    for k in range(4):
        idx_start(k, k, isems[k])
    idx_wait(0, 0, isems[0])
    g_start(0, rows_a, ga)

    def body(j2, carry):
        j = j2 * 4
        for u in range(4):
            cu = j + u
            nslot = (u + 1) % 4
            nbuf = (u + 1) % 2
            if u < 3:
                idx_wait(cu + 1, nslot, isems[nslot])
                g_start(nslot, rows[nbuf], gsems[nbuf])
            else:
                @pl.when(cu + 1 < CPT)
                def _():
                    idx_wait(cu + 1, nslot, isems[nslot])
                    g_start(nslot, rows[nbuf], gsems[nbuf])
            g_wait(u, rows[u % 2], gsems[u % 2])
            scat(u, rows[u % 2])
            @pl.when(cu + 4 < CPT)
            def _():
                idx_start(cu + 4, u, isems[u])
        return carry

    lax.fori_loop(0, L, body, 0)
    for k in range(R):
        cu = 4 * L + k
        if k + 1 < R:
            idx_wait(cu + 1, k + 1, isems[k + 1])
            g_start(k + 1, rows[(k + 1) % 2], gsems[(k + 1) % 2])
        g_wait(k, rows[k % 2], gsems[k % 2])
        scat(k, rows[k % 2])
    plsc.subcore_barrier()
    pltpu.sync_copy(acc.at[pl.ds(base, RPT)], out_hbm.at[c, pl.ds(base, RPT)])


def _sig(v):
    return 1.0 / (1.0 + jnp.exp(-v))


def _tc_dinv_body(deg_ref, out_ref):
    d = deg_ref[0, :, 0:1] + deg_ref[1, :, 0:1] + 1.0
    dv = lax.rsqrt(d)
    out_ref[...] = jnp.broadcast_to(dv, (RB, DD))


_tc_dinv = pl.pallas_call(
    _tc_dinv_body,
    grid=(GRID,),
    in_specs=[pl.BlockSpec((NC, RB, DD), lambda i: (0, i, 0))],
    out_specs=pl.BlockSpec((RB, DD), lambda i: (i, 0)),
    out_shape=jax.ShapeDtypeStruct((NN, DD), jnp.float32),
)


def _tc_prep1_body(x_ref, dinv_ref, w_ref, b_ref, y_ref, s_ref):
    xw = jnp.dot(x_ref[...], w_ref[...], preferred_element_type=jnp.float32)
    dv = dinv_ref[...]
    y_ref[...] = dv * xw
    s_ref[...] = dv * dv * xw + b_ref[...]


_tc_prep1 = pl.pallas_call(
    _tc_prep1_body,
    grid=(GRID,),
    in_specs=[
        pl.BlockSpec((RB, DD), lambda i: (i, 0)),
        pl.BlockSpec((RB, DD), lambda i: (i, 0)),
        pl.BlockSpec((DD, DD), lambda i: (0, 0)),
        pl.BlockSpec((1, DD), lambda i: (0, 0)),
    ],
    out_specs=[
        pl.BlockSpec((RB, DD), lambda i: (i, 0)),
        pl.BlockSpec((RB, DD), lambda i: (i, 0)),
    ],
    out_shape=[
        jax.ShapeDtypeStruct((NN, DD), jnp.float32),
        jax.ShapeDtypeStruct((NN, DD), jnp.float32),
    ],
)


def _tc_mid_body(agg_ref, s_ref, dinv_ref, w_ref, b_ref, y_ref, s2_ref):
    dv = dinv_ref[...]
    h = _sig(dv * (agg_ref[0] + agg_ref[1]) + s_ref[...])
    xw = jnp.dot(h, w_ref[...], preferred_element_type=jnp.float32)
    y_ref[...] = dv * xw
    s2_ref[...] = dv * dv * xw + b_ref[...]


_tc_mid = pl.pallas_call(
    _tc_mid_body,
    grid=(GRID,),
    in_specs=[
        pl.BlockSpec((NC, RB, DD), lambda i: (0, i, 0)),
        pl.BlockSpec((RB, DD), lambda i: (i, 0)),
        pl.BlockSpec((RB, DD), lambda i: (i, 0)),
        pl.BlockSpec((DD, DD), lambda i: (0, 0)),
        pl.BlockSpec((1, DD), lambda i: (0, 0)),
    ],
    out_specs=[
        pl.BlockSpec((RB, DD), lambda i: (i, 0)),
        pl.BlockSpec((RB, DD), lambda i: (i, 0)),
    ],
    out_shape=[
        jax.ShapeDtypeStruct((NN, DD), jnp.float32),
        jax.ShapeDtypeStruct((NN, DD), jnp.float32),
    ],
)


def _tc_final_body(agg_ref, s_ref, dinv_ref, batch_ref, wc1_ref, bc1_ref,
                   wc2_ref, bc2_ref, out_ref, sums_ref, cnts_ref):
    i = pl.program_id(0)

    @pl.when(i == 0)
    def _():
        sums_ref[...] = jnp.zeros((GG, DD), jnp.float32)
        cnts_ref[...] = jnp.zeros((GG, DD), jnp.float32)
        out_ref[...] = jnp.zeros((GG, 1), jnp.float32)

    dv = dinv_ref[...]
    h = _sig(dv * (agg_ref[0] + agg_ref[1]) + s_ref[...])
    ids = batch_ref[0, 0, :]
    oh = (ids[None, :] == lax.broadcasted_iota(jnp.int32, (GG, RB), 0))
    oh = oh.astype(jnp.float32)
    sums_ref[...] += jnp.dot(oh, h, preferred_element_type=jnp.float32)
    cnts_ref[...] += jnp.broadcast_to(
        jnp.sum(oh, axis=1, keepdims=True), (GG, DD))

    @pl.when(i == GRID - 1)
    def _():
        pooled = sums_ref[...] / jnp.maximum(cnts_ref[...], 1.0)
        hc = _sig(jnp.dot(pooled, wc1_ref[...],
                          preferred_element_type=jnp.float32) + bc1_ref[...])
        out_ref[...] = jnp.dot(hc, wc2_ref[...],
                               preferred_element_type=jnp.float32) + bc2_ref[...]


_tc_final = pl.pallas_call(
    _tc_final_body,
    grid=(GRID,),
    in_specs=[
        pl.BlockSpec((NC, RB, DD), lambda i: (0, i, 0)),
        pl.BlockSpec((RB, DD), lambda i: (i, 0)),
        pl.BlockSpec((RB, DD), lambda i: (i, 0)),
        pl.BlockSpec((1, 1, RB), lambda i: (i, 0, 0)),
        pl.BlockSpec((DD, GG), lambda i: (0, 0)),
        pl.BlockSpec((1, GG), lambda i: (0, 0)),
        pl.BlockSpec((GG, 1), lambda i: (0, 0)),
        pl.BlockSpec((1, 1), lambda i: (0, 0)),
    ],
    out_specs=pl.BlockSpec((GG, 1), lambda i: (0, 0)),
    out_shape=jax.ShapeDtypeStruct((GG, 1), jnp.float32),
    scratch_shapes=[
        pltpu.VMEM((GG, DD), jnp.float32),
        pltpu.VMEM((GG, DD), jnp.float32),
    ],
)


def kernel(x, edge_index, batch, num_graphs, W1, b1, W2, b2, Wc1, bc1, Wc2, bc2):
    src = edge_index[0]
    dst = edge_index[1]
    npad = EPAD - EE
    src3 = jnp.concatenate(
        [src, jnp.zeros((npad,), jnp.int32)]).reshape(NW, CPT, CHUNK)
    dst3 = jnp.concatenate(
        [dst, jnp.full((npad,), NN, jnp.int32)]).reshape(NW, CPT, CHUNK)
    ei4 = jnp.stack([src3, dst3], axis=2)
    batch3 = batch.reshape(GRID, 1, RB)
    ones_row = jnp.ones((CHUNK, DD), jnp.float32)
    zero_row = jnp.zeros((CHUNK, DD), jnp.float32)

    deg2 = _sc_deg(dst3, ones_row, zero_row)
    dinvb = _tc_dinv(deg2)
    y1, s1 = _tc_prep1(x, dinvb, W1, b1.reshape(1, DD))
    agg1 = _sc_agg(y1, ei4, zero_row)
    y2, s2 = _tc_mid(agg1, s1, dinvb, W2, b2.reshape(1, DD))
    agg2 = _sc_agg(y2, ei4, zero_row)
    out = _tc_final(agg2, s2, dinvb, batch3, Wc1, bc1.reshape(1, GG),
                    Wc2, bc2.reshape(1, 1))
    return out
